# baseline (speedup 1.0000x reference)
import functools

import jax
import jax.numpy as jnp
from jax.experimental import pallas as pl
from jax.experimental.pallas import tpu as pltpu

_PAD = 6
_K = 25
_INV_K = 1.0 / _K
_BB = 8
_NSLOT = 4


def _window_sum(x, M):
    bb, Lp, C = x.shape
    L = Lp - 16
    s8 = x[:, 0:L] + x[:, 8:L + 8] + x[:, 16:L + 16]
    p2 = s8[:, 0:L - 1] + s8[:, 1:L]
    p4 = p2[:, 0:L - 3] + p2[:, 2:L - 1]
    p8 = p4[:, 0:L - 7] + p4[:, 4:L - 3]
    return p8[:, 3:M + 3] + x[:, 27:M + 27]


def _mavg_kernel(x_hbm, o_hbm, xbuf, obuf, in_sems, out_sems, *, nchunks):
    @pl.when(pl.program_id(0) == 0)
    def _body():
        _mavg_body(x_hbm, o_hbm, xbuf, obuf, in_sems, out_sems,
                   nchunks=nchunks)


def _mavg_body(x_hbm, o_hbm, xbuf, obuf, in_sems, out_sems, *, nchunks):
    bb = _BB
    M = o_hbm.shape[1]
    L = x_hbm.shape[1]

    def _read(k):
        slot = k % _NSLOT
        return pltpu.make_async_copy(
            x_hbm.at[pl.ds(k * bb, bb)],
            xbuf.at[pl.ds(slot * bb, bb), pl.ds(8, L)],
            in_sems.at[slot])

    def _write(k):
        slot = k % _NSLOT
        return pltpu.make_async_copy(
            obuf.at[pl.ds(slot * bb, bb)],
            o_hbm.at[pl.ds(k * bb, bb)],
            out_sems.at[slot])

    for k in range(min(_NSLOT, nchunks)):
        _read(k).start()

    xbuf[:, 0:8] = jnp.zeros((xbuf.shape[0], 8, xbuf.shape[2]), xbuf.dtype)
    xbuf[:, L + 8:L + 16] = jnp.zeros((xbuf.shape[0], 8, xbuf.shape[2]),
                                      xbuf.dtype)

    for k in range(nchunks):
        slot = k % _NSLOT
        _read(k).wait()
        if k >= _NSLOT:
            _write(k - _NSLOT).wait()
        xc = xbuf[pl.ds(slot * bb, bb)]
        obuf[pl.ds(slot * bb, bb)] = _window_sum(xc, M) * jnp.float32(_INV_K)
        _write(k).start()
        if k + _NSLOT < nchunks:
            _read(k + _NSLOT).start()

    for k in range(max(0, nchunks - _NSLOT), nchunks):
        _write(k).wait()


def kernel(x):
    B, L, C = x.shape
    L_pool = (L + 2 * _PAD - _K) // 1 + 1
    M = L_pool - 1

    nchunks = B // _BB
    kfn = functools.partial(_mavg_kernel, nchunks=nchunks)

    return pl.pallas_call(
        kfn,
        out_shape=jax.ShapeDtypeStruct((B, M, C), x.dtype),
        grid=(2,),
        in_specs=[pl.BlockSpec(memory_space=pl.ANY)],
        out_specs=pl.BlockSpec(memory_space=pl.ANY),
        scratch_shapes=[
            pltpu.VMEM((_NSLOT * _BB, L + 16, C), x.dtype),
            pltpu.VMEM((_NSLOT * _BB, M, C), x.dtype),
            pltpu.SemaphoreType.DMA((_NSLOT,)),
            pltpu.SemaphoreType.DMA((_NSLOT,)),
        ],
        compiler_params=pltpu.CompilerParams(
            dimension_semantics=("arbitrary",),
            vmem_limit_bytes=100 * 1024 * 1024),
    )(x)


# speedup vs baseline: 1.0033x; 1.0033x over previous
import functools

import jax
import jax.numpy as jnp
from jax.experimental import pallas as pl
from jax.experimental.pallas import tpu as pltpu

_PAD = 6
_K = 25
_INV_K = 1.0 / _K
_BB = 8
_NSLOT = 4


def _window_sum(x, M):
    bb, Lp, C = x.shape
    L = Lp - 16
    s8 = x[:, 0:L] + x[:, 8:L + 8] + x[:, 16:L + 16]
    p2 = s8[:, 0:L - 1] + s8[:, 1:L]
    p4 = p2[:, 0:L - 3] + p2[:, 2:L - 1]
    p8 = p4[:, 0:L - 7] + p4[:, 4:L - 3]
    return p8[:, 3:M + 3] + x[:, 27:M + 27]


def _mavg_kernel(x_hbm, o_hbm, xbuf, obuf, in_sems, out_sems, *, nchunks):
    bb = _BB
    M = o_hbm.shape[1]
    L = x_hbm.shape[1]

    def _read(k):
        slot = k % _NSLOT
        return pltpu.make_async_copy(
            x_hbm.at[pl.ds(k * bb, bb)],
            xbuf.at[pl.ds(slot * bb, bb), pl.ds(8, L)],
            in_sems.at[slot])

    def _write(k):
        slot = k % _NSLOT
        return pltpu.make_async_copy(
            obuf.at[pl.ds(slot * bb, bb)],
            o_hbm.at[pl.ds(k * bb, bb)],
            out_sems.at[slot])

    for k in range(min(_NSLOT, nchunks)):
        _read(k).start()

    xbuf[:, 0:8] = jnp.zeros((xbuf.shape[0], 8, xbuf.shape[2]), xbuf.dtype)
    xbuf[:, L + 8:L + 16] = jnp.zeros((xbuf.shape[0], 8, xbuf.shape[2]),
                                      xbuf.dtype)

    for k in range(nchunks):
        slot = k % _NSLOT
        _read(k).wait()
        if k >= _NSLOT:
            _write(k - _NSLOT).wait()
        xc = xbuf[pl.ds(slot * bb, bb)]
        obuf[pl.ds(slot * bb, bb)] = _window_sum(xc, M) * jnp.float32(_INV_K)
        _write(k).start()
        if k + _NSLOT < nchunks:
            _read(k + _NSLOT).start()

    for k in range(max(0, nchunks - _NSLOT), nchunks):
        _write(k).wait()


def kernel(x):
    B, L, C = x.shape
    L_pool = (L + 2 * _PAD - _K) // 1 + 1
    M = L_pool - 1

    nchunks = B // _BB
    kfn = functools.partial(_mavg_kernel, nchunks=nchunks)

    return pl.pallas_call(
        kfn,
        out_shape=jax.ShapeDtypeStruct((B, M, C), x.dtype),
        grid=(1,),
        in_specs=[pl.BlockSpec(memory_space=pl.ANY)],
        out_specs=pl.BlockSpec(memory_space=pl.ANY),
        scratch_shapes=[
            pltpu.VMEM((_NSLOT * _BB, L + 16, C), x.dtype),
            pltpu.VMEM((_NSLOT * _BB, M, C), x.dtype),
            pltpu.SemaphoreType.DMA((_NSLOT,)),
            pltpu.SemaphoreType.DMA((_NSLOT,)),
        ],
        compiler_params=pltpu.CompilerParams(
            dimension_semantics=("arbitrary",),
            vmem_limit_bytes=100 * 1024 * 1024),
    )(x)
